# baseline (speedup 1.0000x reference)
import functools
import math

import jax
import jax.numpy as jnp
from jax.experimental import pallas as pl
from jax.experimental.pallas import tpu as pltpu

BM = 512
BK = 1024
BQ = 512
BKV = 1024


def _support_body(x_ref, w_ref, o_ref):
    x = x_ref[...].astype(jnp.bfloat16)
    w = w_ref[...].astype(jnp.bfloat16)
    o_ref[...] = jax.lax.dot_general(
        x, w, (((1,), (0,)), ((), ())),
        preferred_element_type=jnp.float32).astype(jnp.bfloat16)


def _support(x, w):
    m, k = x.shape
    f = w.shape[1]
    return pl.pallas_call(
        _support_body,
        grid=(m // BM,),
        in_specs=[
            pl.BlockSpec((BM, k), lambda i: (i, 0)),
            pl.BlockSpec((k, f), lambda i: (0, 0)),
        ],
        out_specs=pl.BlockSpec((BM, f), lambda i: (i, 0)),
        out_shape=jax.ShapeDtypeStruct((m, f), jnp.bfloat16),
    )(x, w)


def _gc_body(adj_ref, s_ref, b_ref, o_ref, acc_ref, *, nk):
    k = pl.program_id(1)

    @pl.when(k == 0)
    def _():
        acc_ref[...] = jnp.zeros_like(acc_ref)

    a = adj_ref[...].astype(jnp.bfloat16)
    acc_ref[...] += jax.lax.dot_general(
        a, s_ref[...], (((1,), (0,)), ((), ())),
        preferred_element_type=jnp.float32)

    @pl.when(k == nk - 1)
    def _():
        o_ref[...] = jax.nn.sigmoid(
            acc_ref[...] + b_ref[...].astype(jnp.float32)).astype(jnp.bfloat16)


def _gc(adj, s, b):
    n = adj.shape[0]
    f = s.shape[1]
    nk = n // BK
    return pl.pallas_call(
        functools.partial(_gc_body, nk=nk),
        grid=(n // BM, nk),
        in_specs=[
            pl.BlockSpec((BM, BK), lambda i, k: (i, k)),
            pl.BlockSpec((BK, f), lambda i, k: (k, 0)),
            pl.BlockSpec((1, f), lambda i, k: (0, 0)),
        ],
        out_specs=pl.BlockSpec((BM, f), lambda i, k: (i, 0)),
        out_shape=jax.ShapeDtypeStruct((n, f), jnp.bfloat16),
        scratch_shapes=[pltpu.VMEM((BM, f), jnp.float32)],
        compiler_params=pltpu.CompilerParams(
            dimension_semantics=("parallel", "arbitrary")),
    )(adj, s, b)


def _attn_body(q_ref, k_ref, v_ref, o_ref, acc_ref, m_ref, l_ref, *,
               nkv, scale):
    j = pl.program_id(1)

    @pl.when(j == 0)
    def _():
        acc_ref[...] = jnp.zeros_like(acc_ref)
        m_ref[...] = jnp.full_like(m_ref, -1e30)
        l_ref[...] = jnp.zeros_like(l_ref)

    q = q_ref[...]
    s = jax.lax.dot_general(
        q, k_ref[...], (((1,), (1,)), ((), ())),
        preferred_element_type=jnp.float32) * scale
    m_prev = m_ref[...]
    m_cur = jnp.maximum(m_prev, jnp.max(s, axis=1, keepdims=True))
    alpha = jnp.exp(m_prev - m_cur)
    p = jnp.exp(s - m_cur)
    l_ref[...] = l_ref[...] * alpha + jnp.sum(p, axis=1, keepdims=True)
    acc_ref[...] = acc_ref[...] * alpha + jax.lax.dot_general(
        p.astype(jnp.bfloat16), v_ref[...], (((1,), (0,)), ((), ())),
        preferred_element_type=jnp.float32)
    m_ref[...] = m_cur

    @pl.when(j == nkv - 1)
    def _():
        o_ref[...] = q.astype(jnp.float32) + acc_ref[...] / l_ref[...]


def _attn(q, k, v, scale):
    n, d = q.shape
    nkv = n // BKV
    return pl.pallas_call(
        functools.partial(_attn_body, nkv=nkv, scale=scale),
        grid=(n // BQ, nkv),
        in_specs=[
            pl.BlockSpec((BQ, d), lambda i, j: (i, 0)),
            pl.BlockSpec((BKV, d), lambda i, j: (j, 0)),
            pl.BlockSpec((BKV, d), lambda i, j: (j, 0)),
        ],
        out_specs=pl.BlockSpec((BQ, d), lambda i, j: (i, 0)),
        out_shape=jax.ShapeDtypeStruct((n, d), jnp.float32),
        scratch_shapes=[
            pltpu.VMEM((BQ, d), jnp.float32),
            pltpu.VMEM((BQ, 1), jnp.float32),
            pltpu.VMEM((BQ, 1), jnp.float32),
        ],
        compiler_params=pltpu.CompilerParams(
            dimension_semantics=("parallel", "arbitrary")),
    )(q, k, v)


def kernel(x1, adj1, x2, adj2, W1, b1, W2, b2):
    b1r = b1.reshape(1, -1)
    b2r = b2.reshape(1, -1)

    def tower(x, adj):
        h = _gc(adj, _support(x, W1), b1r)
        return _gc(adj, _support(h, W2), b2r)

    h1 = tower(x1, adj1)
    h2 = tower(x2, adj2)
    scale = 1.0 / math.sqrt(h1.shape[1])
    o1 = _attn(h1, h2, h2, scale)
    o2 = _attn(h2, h1, h1, scale)
    return (o1, o2)
